# baseline (speedup 1.0000x reference)
import functools

import jax
import jax.numpy as jnp
from jax import lax
from jax.experimental import pallas as pl
from jax.experimental.pallas import tpu as pltpu
from jax.experimental.pallas import tpu_sc as plsc

N_NODES = 10000
N_EDGES = 320000
D_FEAT = 128
H_DIM = 512
EMB_DIM = 128

NC = 2
NS = 16
CH = 128
K = (-(-N_EDGES // (NC * NS * CH)) + 1) & ~1
E_PAD = NC * NS * K * CH
ROWS_PER_TILE = (N_NODES + 1 + NS - 1) // NS // 2 * 2
N_PAD = ROWS_PER_TILE * NS
_BN = 1024

_mesh = plsc.VectorSubcoreMesh(core_axis_name="c", subcore_axis_name="s")


@functools.partial(
    pl.kernel,
    out_type=(
        jax.ShapeDtypeStruct((NC, N_PAD, D_FEAT), jnp.bfloat16),
        jax.ShapeDtypeStruct((NC, N_PAD, 16), jnp.float32),
    ),
    mesh=_mesh,
    compiler_params=pltpu.CompilerParams(use_tc_tiling_on_sc=False),
    scratch_types=[
        pltpu.VMEM((K, CH), jnp.int32),
        pltpu.VMEM((K, CH), jnp.int32),
        pltpu.VMEM((2, CH, D_FEAT), jnp.bfloat16),
        pltpu.VMEM((CH, 16), jnp.float32),
        pltpu.VMEM_SHARED((N_PAD, D_FEAT), jnp.bfloat16),
        pltpu.VMEM_SHARED((N_PAD, 16), jnp.float32),
        pltpu.SemaphoreType.DMA,
        pltpu.SemaphoreType.DMA,
        pltpu.SemaphoreType.DMA,
    ],
)
def _sc_segment_sum(xb_hbm, src_hbm, dst_hbm, zb_hbm, z16_hbm, ones_hbm,
                    agg_hbm, deg_hbm,
                    src_v, dst_v, rows_v, ones_v, agg_sh, deg_sh,
                    gsem0, gsem1, osem):
    c = lax.axis_index("c")
    s = lax.axis_index("s")
    pltpu.sync_copy(src_hbm.at[c, s], src_v)
    pltpu.sync_copy(dst_hbm.at[c, s], dst_v)
    pltpu.sync_copy(ones_hbm, ones_v)
    r0 = s * ROWS_PER_TILE
    pltpu.sync_copy(zb_hbm.at[pl.ds(r0, ROWS_PER_TILE)],
                    agg_sh.at[pl.ds(r0, ROWS_PER_TILE)])
    pltpu.sync_copy(z16_hbm.at[pl.ds(r0, ROWS_PER_TILE)],
                    deg_sh.at[pl.ds(r0, ROWS_PER_TILE)])
    plsc.subcore_barrier()

    def _gather_start(j, b, sem):
        pltpu.async_copy(xb_hbm.at[src_v.at[j]], rows_v.at[b], sem)

    def _gather_wait(j, b, sem):
        pltpu.make_async_copy(xb_hbm.at[src_v.at[j]], rows_v.at[b], sem).wait()

    def _ones_start(j):
        pltpu.async_copy(ones_v, deg_sh.at[dst_v.at[j]], osem, add=True)

    def _ones_wait(j):
        pltpu.make_async_copy(ones_v, deg_sh.at[dst_v.at[j]], osem).wait()

    _gather_start(0, 0, gsem0)

    @pl.loop(0, K, step=2)
    def _(j):
        _gather_start(j + 1, 1, gsem1)
        _gather_wait(j, 0, gsem0)
        pltpu.sync_copy(rows_v.at[0], agg_sh.at[dst_v.at[j]], add=True)

        @pl.when(j >= 2)
        def _():
            _ones_wait(j - 1)

        _ones_start(j)

        @pl.when(j + 2 < K)
        def _():
            _gather_start(j + 2, 0, gsem0)

        _gather_wait(j + 1, 1, gsem1)
        pltpu.sync_copy(rows_v.at[1], agg_sh.at[dst_v.at[j + 1]], add=True)
        _ones_wait(j)
        _ones_start(j + 1)

    _ones_wait(K - 1)
    plsc.subcore_barrier()
    pltpu.sync_copy(agg_sh.at[pl.ds(r0, ROWS_PER_TILE)],
                    agg_hbm.at[c, pl.ds(r0, ROWS_PER_TILE)])
    pltpu.sync_copy(deg_sh.at[pl.ds(r0, ROWS_PER_TILE)],
                    deg_hbm.at[c, pl.ds(r0, ROWS_PER_TILE)])


def _dot(a, b):
    return jnp.dot(a, b, preferred_element_type=jnp.float32)


def _tc_body(x_ref, agg_ref, deg_ref, wl1, bl1, wl2, bl2, wg1t, bg1, wg1b,
             wg2, bg2, el_ref, eg_ref):
    x = x_ref[...]
    hl = jnp.maximum(_dot(x, wl1[...]) + bl1[...], 0.0)
    el_ref[...] = _dot(hl, wl2[...]) + bl2[...]
    a = agg_ref[...]
    ag = a[0].astype(jnp.float32) + a[1].astype(jnp.float32)
    d = deg_ref[...]
    inv = 1.0 / jnp.maximum(d[0, :, 0:1] + d[1, :, 0:1], 1.0)
    mean = ag * inv
    h = jnp.maximum(_dot(x, wg1t[...]) + bg1[...] + _dot(mean, wg1b[...]), 0.0)
    eg_ref[...] = _dot(h, wg2[...]) + bg2[...]


def _full(shape):
    return pl.BlockSpec(shape, lambda i: tuple(0 for _ in shape))


def kernel(x, edge_index, Wg1, bg1, Wg2, bg2, Wl1, bl1, Wl2, bl2):
    src = edge_index[0].astype(jnp.int32)
    dst = edge_index[1].astype(jnp.int32)
    src3 = jnp.concatenate(
        [src, jnp.zeros((E_PAD - N_EDGES,), jnp.int32)]).reshape(NC, NS, K, CH)
    dst3 = jnp.concatenate(
        [dst, jnp.full((E_PAD - N_EDGES,), N_NODES, jnp.int32)]).reshape(NC, NS, K, CH)
    xb = x.astype(jnp.bfloat16)
    zb = jnp.zeros((N_PAD, D_FEAT), jnp.bfloat16)
    z16 = jnp.zeros((N_PAD, 16), jnp.float32)
    ones = jnp.ones((CH, 16), jnp.float32)

    agg2, deg2 = _sc_segment_sum(xb, src3, dst3, zb, z16, ones)

    n_blocks = -(-N_NODES // _BN)
    el, eg = pl.pallas_call(
        _tc_body,
        grid=(n_blocks,),
        in_specs=[
            pl.BlockSpec((_BN, D_FEAT), lambda i: (i, 0)),
            pl.BlockSpec((NC, _BN, D_FEAT), lambda i: (0, i, 0)),
            pl.BlockSpec((NC, _BN, 16), lambda i: (0, i, 0)),
            _full((D_FEAT, H_DIM)),
            _full((1, H_DIM)),
            _full((H_DIM, EMB_DIM)),
            _full((1, EMB_DIM)),
            _full((D_FEAT, H_DIM)),
            _full((1, H_DIM)),
            _full((D_FEAT, H_DIM)),
            _full((H_DIM, EMB_DIM)),
            _full((1, EMB_DIM)),
        ],
        out_specs=[
            pl.BlockSpec((_BN, EMB_DIM), lambda i: (i, 0)),
            pl.BlockSpec((_BN, EMB_DIM), lambda i: (i, 0)),
        ],
        out_shape=[
            jax.ShapeDtypeStruct((N_NODES, EMB_DIM), jnp.float32),
            jax.ShapeDtypeStruct((N_NODES, EMB_DIM), jnp.float32),
        ],
    )(x, agg2, deg2, Wl1, bl1.reshape(1, H_DIM), Wl2, bl2.reshape(1, EMB_DIM),
      Wg1[:D_FEAT], bg1.reshape(1, H_DIM), Wg1[D_FEAT:], Wg2,
      bg2.reshape(1, EMB_DIM))

    return eg, el


# speedup vs baseline: 1.1749x; 1.1749x over previous
import functools

import jax
import jax.numpy as jnp
from jax import lax
from jax.experimental import pallas as pl
from jax.experimental.pallas import tpu as pltpu
from jax.experimental.pallas import tpu_sc as plsc

N_NODES = 10000
N_EDGES = 320000
D_FEAT = 128
H_DIM = 512
EMB_DIM = 128

NC = 2
NS = 16
DH = D_FEAT // NC
CH = 128
K = (-(-N_EDGES // (NS * CH)) + 1) & ~1
E_PAD = NS * K * CH
_BN = 1024
N_PAD = -(-(N_NODES + 1) // _BN) * _BN
ROWS_PER_TILE = N_PAD // NS

_mesh = plsc.VectorSubcoreMesh(core_axis_name="c", subcore_axis_name="s")


@functools.partial(
    pl.kernel,
    out_type=(
        jax.ShapeDtypeStruct((NC, N_PAD, DH), jnp.float32),
        jax.ShapeDtypeStruct((NC, N_PAD, 16), jnp.float32),
    ),
    mesh=_mesh,
    compiler_params=pltpu.CompilerParams(use_tc_tiling_on_sc=False),
    scratch_types=[
        pltpu.VMEM((K, CH), jnp.int32),
        pltpu.VMEM((K, CH), jnp.int32),
        pltpu.VMEM((2, CH, DH), jnp.float32),
        pltpu.VMEM((CH, 16), jnp.float32),
        pltpu.VMEM_SHARED((N_PAD, DH), jnp.float32),
        pltpu.VMEM_SHARED((N_PAD, 16), jnp.float32),
        pltpu.SemaphoreType.DMA,
        pltpu.SemaphoreType.DMA,
        pltpu.SemaphoreType.DMA,
    ],
)
def _sc_segment_sum(xs_hbm, src_hbm, dst_hbm, z64_hbm, z16_hbm, ones_hbm,
                    agg_hbm, deg_hbm,
                    src_v, dst_v, rows_v, ones_v, agg_sh, deg_sh,
                    gsem0, gsem1, osem):
    c = lax.axis_index("c")
    s = lax.axis_index("s")
    pltpu.sync_copy(src_hbm.at[s], src_v)
    pltpu.sync_copy(dst_hbm.at[s], dst_v)
    pltpu.sync_copy(ones_hbm, ones_v)
    r0 = s * ROWS_PER_TILE
    pltpu.sync_copy(z64_hbm.at[pl.ds(r0, ROWS_PER_TILE)],
                    agg_sh.at[pl.ds(r0, ROWS_PER_TILE)])
    pltpu.sync_copy(z16_hbm.at[pl.ds(r0, ROWS_PER_TILE)],
                    deg_sh.at[pl.ds(r0, ROWS_PER_TILE)])
    plsc.subcore_barrier()

    def _gather_start(j, b, sem):
        pltpu.async_copy(xs_hbm.at[c].at[src_v.at[j]], rows_v.at[b], sem)

    def _gather_wait(j, b, sem):
        pltpu.make_async_copy(xs_hbm.at[c].at[src_v.at[j]], rows_v.at[b],
                              sem).wait()

    def _ones_start(j):
        pltpu.async_copy(ones_v, deg_sh.at[dst_v.at[j]], osem, add=True)

    def _ones_wait(j):
        pltpu.make_async_copy(ones_v, deg_sh.at[dst_v.at[j]], osem).wait()

    _gather_start(0, 0, gsem0)

    @pl.loop(0, K, step=2)
    def _(j):
        _gather_start(j + 1, 1, gsem1)
        _gather_wait(j, 0, gsem0)
        pltpu.sync_copy(rows_v.at[0], agg_sh.at[dst_v.at[j]], add=True)

        @pl.when((c == 0) & (j >= 2))
        def _():
            _ones_wait(j - 2)

        @pl.when(c == 0)
        def _():
            _ones_start(j)

        @pl.when(j + 2 < K)
        def _():
            _gather_start(j + 2, 0, gsem0)

        _gather_wait(j + 1, 1, gsem1)
        pltpu.sync_copy(rows_v.at[1], agg_sh.at[dst_v.at[j + 1]], add=True)

        @pl.when((c == 1) & (j >= 2))
        def _():
            _ones_wait(j - 1)

        @pl.when(c == 1)
        def _():
            _ones_start(j + 1)

    @pl.when(c == 0)
    def _():
        _ones_wait(K - 2)

    @pl.when(c == 1)
    def _():
        _ones_wait(K - 1)

    plsc.subcore_barrier()
    pltpu.sync_copy(agg_sh.at[pl.ds(r0, ROWS_PER_TILE)],
                    agg_hbm.at[c, pl.ds(r0, ROWS_PER_TILE)])
    pltpu.sync_copy(deg_sh.at[pl.ds(r0, ROWS_PER_TILE)],
                    deg_hbm.at[c, pl.ds(r0, ROWS_PER_TILE)])


def _dot(a, b):
    return jnp.dot(a, b, preferred_element_type=jnp.float32)


def _bf(v):
    return v.astype(jnp.bfloat16)


def _tc_local_body(x_ref, wl1, bl1, wl2, bl2, wg1t, bg1, el_ref, pre_ref):
    x = _bf(x_ref[...])
    hl = jnp.maximum(_dot(x, wl1[...]) + bl1[...], 0.0)
    el_ref[...] = _dot(_bf(hl), wl2[...]) + bl2[...]
    pre_ref[...] = _dot(x, wg1t[...]) + bg1[...]


def _tc_global_body(pre_ref, agg_ref, deg_ref, wg1b0, wg1b1, wg2, bg2, eg_ref):
    a = agg_ref[...]
    d = deg_ref[...]
    inv = 1.0 / jnp.maximum(d[0, :, 0:1] + d[1, :, 0:1], 1.0)
    h = jnp.maximum(pre_ref[...] + _dot(_bf(a[0] * inv), wg1b0[...])
                    + _dot(_bf(a[1] * inv), wg1b1[...]), 0.0)
    eg_ref[...] = _dot(_bf(h), wg2[...]) + bg2[...]


def _full(shape):
    return pl.BlockSpec(shape, lambda i: tuple(0 for _ in shape))


def kernel(x, edge_index, Wg1, bg1, Wg2, bg2, Wl1, bl1, Wl2, bl2):
    src = edge_index[0].astype(jnp.int32)
    dst = edge_index[1].astype(jnp.int32)
    src2 = jnp.concatenate(
        [src, jnp.zeros((E_PAD - N_EDGES,), jnp.int32)]).reshape(NS, K, CH)
    dst2 = jnp.concatenate(
        [dst, jnp.full((E_PAD - N_EDGES,), N_NODES, jnp.int32)]).reshape(NS, K, CH)
    xs = jnp.stack([x[:, :DH], x[:, DH:]])
    z64 = jnp.zeros((N_PAD, DH), jnp.float32)
    z16 = jnp.zeros((N_PAD, 16), jnp.float32)
    ones = jnp.ones((CH, 16), jnp.float32)

    agg2, deg = _sc_segment_sum(xs, src2, dst2, z64, z16, ones)

    n_blocks = -(-N_NODES // _BN)
    el, pre = pl.pallas_call(
        _tc_local_body,
        grid=(n_blocks,),
        in_specs=[
            pl.BlockSpec((_BN, D_FEAT), lambda i: (i, 0)),
            _full((D_FEAT, H_DIM)),
            _full((1, H_DIM)),
            _full((H_DIM, EMB_DIM)),
            _full((1, EMB_DIM)),
            _full((D_FEAT, H_DIM)),
            _full((1, H_DIM)),
        ],
        out_specs=[
            pl.BlockSpec((_BN, EMB_DIM), lambda i: (i, 0)),
            pl.BlockSpec((_BN, H_DIM), lambda i: (i, 0)),
        ],
        out_shape=[
            jax.ShapeDtypeStruct((N_NODES, EMB_DIM), jnp.float32),
            jax.ShapeDtypeStruct((N_NODES, H_DIM), jnp.float32),
        ],
    )(x, Wl1.astype(jnp.bfloat16), bl1.reshape(1, H_DIM),
      Wl2.astype(jnp.bfloat16), bl2.reshape(1, EMB_DIM),
      Wg1[:D_FEAT].astype(jnp.bfloat16), bg1.reshape(1, H_DIM))

    eg = pl.pallas_call(
        _tc_global_body,
        grid=(n_blocks,),
        in_specs=[
            pl.BlockSpec((_BN, H_DIM), lambda i: (i, 0)),
            pl.BlockSpec((NC, _BN, DH), lambda i: (0, i, 0)),
            pl.BlockSpec((NC, _BN, 16), lambda i: (0, i, 0)),
            _full((DH, H_DIM)),
            _full((DH, H_DIM)),
            _full((H_DIM, EMB_DIM)),
            _full((1, EMB_DIM)),
        ],
        out_specs=pl.BlockSpec((_BN, EMB_DIM), lambda i: (i, 0)),
        out_shape=jax.ShapeDtypeStruct((N_NODES, EMB_DIM), jnp.float32),
    )(pre, agg2, deg, Wg1[D_FEAT:D_FEAT + DH].astype(jnp.bfloat16),
      Wg1[D_FEAT + DH:].astype(jnp.bfloat16), Wg2.astype(jnp.bfloat16),
      bg2.reshape(1, EMB_DIM))

    return eg, el
